# baseline (speedup 1.0000x reference)
import functools

import jax
import jax.numpy as jnp
from jax import lax
from jax.experimental import pallas as pl
from jax.experimental.pallas import tpu as pltpu
from jax.experimental.pallas import tpu_sc as plsc

N = 10000
E = 320000
D = 128
G = 128
DOUT = 10

NC, NS = 2, 16
NW = NC * NS
EPW = E // NW
KCH = 80
EPW_PAD = KCH * 128
EPAD = NW * EPW_PAD
NPAD = 10240
RPT = NPAD // NS
BLK = 512
NBLK = NPAD // BLK

_MESH = plsc.VectorSubcoreMesh(
    core_axis_name="c", subcore_axis_name="s", num_cores=NC, num_subcores=NS)


def _fill_rows(ref, nrows, value):
  width = ref.shape[1]

  @pl.loop(jnp.int32(0), jnp.int32(nrows))
  def _(i):
    for c in range(width // 16):
      ref[i, pl.ds(c * 16, 16)] = jnp.full((16,), value, jnp.float32)


@functools.partial(
    pl.kernel,
    out_type=jax.ShapeDtypeStruct((NC * NPAD, 16), jnp.float32),
    mesh=_MESH,
    scratch_types=[
        pltpu.VMEM((KCH, 128), jnp.int32),
        pltpu.VMEM((128, 16), jnp.float32),
        pltpu.VMEM_SHARED((NPAD, 16), jnp.float32),
    ],
)
def _sc_degree(dst_hbm, out_hbm, dstv, ones_v, acc):
  cid = lax.axis_index("c").astype(jnp.int32)
  sid = lax.axis_index("s").astype(jnp.int32)
  wid = sid * jnp.int32(NC) + cid

  _fill_rows(ones_v, 128, 0.0)
  for q in range(RPT // 128):
    pltpu.sync_copy(ones_v, acc.at[pl.ds(sid * jnp.int32(RPT) + jnp.int32(q * 128), 128)])
  _fill_rows(ones_v, 128, 1.0)
  plsc.subcore_barrier()

  pltpu.sync_copy(dst_hbm.at[wid], dstv)

  @pl.loop(jnp.int32(0), jnp.int32(KCH))
  def _(j):
    pltpu.sync_copy(ones_v, acc.at[dstv.at[j]], add=True)
  plsc.subcore_barrier()
  pltpu.sync_copy(acc.at[pl.ds(sid * jnp.int32(RPT), RPT)],
                  out_hbm.at[pl.ds(cid * jnp.int32(NPAD) + sid * jnp.int32(RPT), RPT)])


@functools.partial(
    pl.kernel,
    out_type=jax.ShapeDtypeStruct((NC * NPAD, D), jnp.float32),
    mesh=_MESH,
    scratch_types=[
        pltpu.VMEM((4, 128), jnp.int32),
        pltpu.VMEM((KCH, 128), jnp.int32),
        pltpu.VMEM((128, D), jnp.float32),
        pltpu.VMEM((128, D), jnp.float32),
        pltpu.VMEM_SHARED((NPAD, D), jnp.float32),
        pltpu.SemaphoreType.DMA,
        pltpu.SemaphoreType.DMA,
        pltpu.SemaphoreType.DMA,
    ],
)
def _sc_scatter(hs_hbm, src_hbm, dst_hbm, out_hbm,
                srcb, dstv, rows0, rows1, acc, sg0, sg1, si):
  cid = lax.axis_index("c").astype(jnp.int32)
  sid = lax.axis_index("s").astype(jnp.int32)
  wid = sid * jnp.int32(NC) + cid

  _fill_rows(rows0, 128, 0.0)
  for q in range(RPT // 128):
    pltpu.sync_copy(rows0, acc.at[pl.ds(sid * jnp.int32(RPT) + jnp.int32(q * 128), 128)])
  plsc.subcore_barrier()

  pltpu.sync_copy(dst_hbm.at[wid], dstv)
  pltpu.sync_copy(src_hbm.at[wid, pl.ds(0, 4)], srcb)
  pltpu.async_copy(hs_hbm.at[srcb.at[jnp.int32(0)]], rows0, sg0)
  pltpu.async_copy(hs_hbm.at[srcb.at[jnp.int32(1)]], rows1, sg1)

  @pl.loop(jnp.int32(0), jnp.int32(KCH), step=jnp.int32(2))
  def _(j):
    s0 = j & jnp.int32(3)
    pltpu.make_async_copy(hs_hbm.at[srcb.at[s0]], rows0, sg0).wait()
    pltpu.sync_copy(rows0, acc.at[dstv.at[j]], add=True)
    pltpu.make_async_copy(hs_hbm.at[srcb.at[s0 + jnp.int32(1)]], rows1, sg1).wait()
    pltpu.sync_copy(rows1, acc.at[dstv.at[j + jnp.int32(1)]], add=True)

    @pl.when(j + jnp.int32(4) < jnp.int32(KCH))
    def _():
      pltpu.async_copy(src_hbm.at[wid, pl.ds(j + jnp.int32(4), 2)],
                       srcb.at[pl.ds(s0, 2)], si)

    @pl.when(j + jnp.int32(2) < jnp.int32(KCH))
    def _():
      s2 = (j + jnp.int32(2)) & jnp.int32(3)

      @pl.when(j >= jnp.int32(2))
      def _():
        pltpu.make_async_copy(src_hbm.at[wid, pl.ds(j + jnp.int32(2), 2)],
                              srcb.at[pl.ds(s2, 2)], si).wait()

      pltpu.async_copy(hs_hbm.at[srcb.at[s2]], rows0, sg0)
      pltpu.async_copy(hs_hbm.at[srcb.at[s2 + jnp.int32(1)]], rows1, sg1)

  plsc.subcore_barrier()
  pltpu.sync_copy(acc.at[pl.ds(sid * jnp.int32(RPT), RPT)],
                  out_hbm.at[pl.ds(cid * jnp.int32(NPAD) + sid * jnp.int32(RPT), RPT)])


def _dot(a, b):
  return jnp.dot(a, b, precision=lax.Precision.HIGHEST,
                 preferred_element_type=jnp.float32)


def _tc_first_body(x_ref, w_ref, d0_ref, d1_ref, hs_ref, dinv_ref):
  deg = 1.0 + d0_ref[:, 0:1] + d1_ref[:, 0:1]
  dinv = lax.rsqrt(deg)
  hs_ref[...] = dinv * _dot(x_ref[...], w_ref[...])
  dinv_ref[...] = jnp.broadcast_to(dinv, (BLK, D))


def _tc_first(x_pad, W1, d0, d1):
  row = pl.BlockSpec((BLK, D), lambda i: (i, jnp.int32(0)))
  deg_spec = pl.BlockSpec((BLK, 16), lambda i: (i, jnp.int32(0)))
  full = pl.BlockSpec((D, D), lambda i: (jnp.int32(0), jnp.int32(0)))
  return pl.pallas_call(
      _tc_first_body,
      grid=(NBLK,),
      in_specs=[row, full, deg_spec, deg_spec],
      out_specs=[row, row],
      out_shape=[jax.ShapeDtypeStruct((NPAD, D), jnp.float32),
                 jax.ShapeDtypeStruct((NPAD, D), jnp.float32)],
  )(x_pad, W1, d0, d1)


def _tc_mid_body(hs_ref, p0_ref, p1_ref, dinv_ref, b_ref, w_ref, out_ref):
  agg = dinv_ref[...] * (p0_ref[...] + p1_ref[...] + hs_ref[...])
  h = jax.nn.relu(agg + b_ref[...])
  out_ref[...] = dinv_ref[...] * _dot(h, w_ref[...])


def _tc_mid(hs, p0, p1, dinv, b_prev, W_next):
  row = pl.BlockSpec((BLK, D), lambda i: (i, jnp.int32(0)))
  full = pl.BlockSpec((D, D), lambda i: (jnp.int32(0), jnp.int32(0)))
  bspec = pl.BlockSpec((1, D), lambda i: (jnp.int32(0), jnp.int32(0)))
  return pl.pallas_call(
      _tc_mid_body,
      grid=(NBLK,),
      in_specs=[row, row, row, row, bspec, full],
      out_specs=row,
      out_shape=jax.ShapeDtypeStruct((NPAD, D), jnp.float32),
  )(hs, p0, p1, dinv, b_prev, W_next)


def _tc_final_body(hs_ref, p0_ref, p1_ref, dinv_ref, b_ref, batch_ref,
                   wl_ref, bl_ref, out_ref, pooled, counts):
  i = pl.program_id(0)
  agg = dinv_ref[...] * (p0_ref[...] + p1_ref[...] + hs_ref[...])
  h = jax.nn.relu(agg + b_ref[...])
  gids = lax.broadcasted_iota(jnp.int32, (BLK, G), 1)
  onehot = (batch_ref[...] == gids).astype(jnp.float32)

  @pl.when(i == 0)
  def _():
    pooled[...] = jnp.zeros((G, D), jnp.float32)
    counts[...] = jnp.zeros((G, D), jnp.float32)

  contract = (((0,), (0,)), ((), ()))
  pooled[...] += lax.dot_general(onehot, h, contract,
                                 precision=lax.Precision.HIGHEST,
                                 preferred_element_type=jnp.float32)
  counts[...] += lax.dot_general(onehot, jnp.ones((BLK, D), jnp.float32),
                                 contract,
                                 precision=lax.Precision.HIGHEST,
                                 preferred_element_type=jnp.float32)

  @pl.when(i == NBLK - 1)
  def _():
    mean = pooled[...] / jnp.maximum(counts[...], 1.0)
    out_ref[...] = _dot(mean, wl_ref[...]) + bl_ref[...]


def _tc_final(hs, p0, p1, dinv, b_prev, batch2d, Wl_pad, bl_pad):
  row = pl.BlockSpec((BLK, D), lambda i: (i, jnp.int32(0)))
  full = pl.BlockSpec((D, D), lambda i: (jnp.int32(0), jnp.int32(0)))
  bspec = pl.BlockSpec((1, D), lambda i: (jnp.int32(0), jnp.int32(0)))
  bat = pl.BlockSpec((BLK, 1), lambda i: (i, jnp.int32(0)))
  outs = pl.BlockSpec((G, D), lambda i: (jnp.int32(0), jnp.int32(0)))
  return pl.pallas_call(
      _tc_final_body,
      grid=(NBLK,),
      in_specs=[row, row, row, row, bspec, bat, full, bspec],
      out_specs=outs,
      out_shape=jax.ShapeDtypeStruct((G, D), jnp.float32),
      scratch_shapes=[pltpu.VMEM((G, D), jnp.float32),
                      pltpu.VMEM((G, D), jnp.float32)],
      compiler_params=pltpu.CompilerParams(
          dimension_semantics=("arbitrary",)),
  )(hs, p0, p1, dinv, b_prev, batch2d, Wl_pad, bl_pad)


def kernel(x, edge_index, batch, W1, b1, W2, b2, W3, b3, Wl, bl):
  src = edge_index[0].astype(jnp.int32)
  dst = edge_index[1].astype(jnp.int32)
  pad_idx = jnp.full((EPAD - E,), N, jnp.int32)
  src_sh = jnp.concatenate([src, pad_idx]).reshape(NW, KCH, 128)
  dst_sh = jnp.concatenate([dst, pad_idx]).reshape(NW, KCH, 128)

  x_pad = jnp.zeros((NPAD, D), jnp.float32).at[:N].set(x)
  batch2d = jnp.concatenate(
      [batch.astype(jnp.int32), jnp.full((NPAD - N,), G, jnp.int32)]
  ).reshape(NPAD, 1)
  b1r, b2r, b3r = b1.reshape(1, D), b2.reshape(1, D), b3.reshape(1, D)
  Wl_pad = jnp.zeros((D, D), jnp.float32).at[:, :DOUT].set(Wl)
  bl_pad = jnp.zeros((1, D), jnp.float32).at[0, :DOUT].set(bl)

  degp = _sc_degree(dst_sh)
  d0, d1 = degp[:NPAD], degp[NPAD:]

  hs1, dinv = _tc_first(x_pad, W1, d0, d1)
  p1 = _sc_scatter(hs1, src_sh, dst_sh)
  hs2 = _tc_mid(hs1, p1[:NPAD], p1[NPAD:], dinv, b1r, W2)
  p2 = _sc_scatter(hs2, src_sh, dst_sh)
  hs3 = _tc_mid(hs2, p2[:NPAD], p2[NPAD:], dinv, b2r, W3)
  p3 = _sc_scatter(hs3, src_sh, dst_sh)
  outp = _tc_final(hs3, p3[:NPAD], p3[NPAD:], dinv, b3r, batch2d,
                   Wl_pad, bl_pad)
  return outp[:G, :DOUT]


# speedup vs baseline: 1.6161x; 1.6161x over previous
import functools

import jax
import jax.numpy as jnp
from jax import lax
from jax.experimental import pallas as pl
from jax.experimental.pallas import tpu as pltpu
from jax.experimental.pallas import tpu_sc as plsc

N = 10000
E = 320000
D = 128
G = 128
DOUT = 10

NC, NS = 2, 16
NW = NC * NS
KC0 = 58
KC1 = 99
KCH = 99
E0 = NS * KC0 * 128
ECAP = NS * (KC0 + KC1) * 128
NPAD = 10240
RPT = NPAD // NS
BLK = 512
NBLK = NPAD // BLK

_MESH = plsc.VectorSubcoreMesh(
    core_axis_name="c", subcore_axis_name="s", num_cores=NC, num_subcores=NS)


def _fill_rows(ref, nrows, value):
  width = ref.shape[1]

  @pl.loop(jnp.int32(0), jnp.int32(nrows))
  def _(i):
    for c in range(width // 16):
      ref[i, pl.ds(c * 16, 16)] = jnp.full((16,), value, jnp.float32)


@functools.partial(
    pl.kernel,
    out_type=jax.ShapeDtypeStruct((NC * NPAD, 16), jnp.float32),
    mesh=_MESH,
    scratch_types=[
        pltpu.VMEM((KCH, 128), jnp.int32),
        pltpu.VMEM((128, 16), jnp.float32),
        pltpu.VMEM_SHARED((NPAD, 16), jnp.float32),
    ],
)
def _sc_degree(dst_hbm, out_hbm, dstv, ones_v, acc):
  cid = lax.axis_index("c").astype(jnp.int32)
  sid = lax.axis_index("s").astype(jnp.int32)
  wid = sid * jnp.int32(NC) + cid

  _fill_rows(ones_v, 128, 0.0)
  for q in range(RPT // 128):
    pltpu.sync_copy(ones_v, acc.at[pl.ds(sid * jnp.int32(RPT) + jnp.int32(q * 128), 128)])
  _fill_rows(ones_v, 128, 1.0)
  plsc.subcore_barrier()

  pltpu.sync_copy(dst_hbm.at[wid], dstv)
  kc = jnp.where(cid == 0, jnp.int32(KC0), jnp.int32(KC1))

  @pl.loop(jnp.int32(0), kc)
  def _(j):
    pltpu.sync_copy(ones_v, acc.at[dstv.at[j]], add=True)
  plsc.subcore_barrier()
  pltpu.sync_copy(acc.at[pl.ds(sid * jnp.int32(RPT), RPT)],
                  out_hbm.at[pl.ds(cid * jnp.int32(NPAD) + sid * jnp.int32(RPT), RPT)])


@functools.partial(
    pl.kernel,
    out_type=jax.ShapeDtypeStruct((NC * NPAD, D), jnp.float32),
    mesh=_MESH,
    scratch_types=[
        pltpu.VMEM((KCH, 128), jnp.int32),
        pltpu.VMEM((KCH, 128), jnp.int32),
        pltpu.VMEM((128, D), jnp.float32),
        pltpu.VMEM_SHARED((NPAD, D), jnp.float32),
        pltpu.SemaphoreType.DMA,
    ],
)
def _sc_scatter(hs_hbm, src_hbm, dst_hbm, out_hbm, srcv, dstv, rows, acc, sem):
  cid = lax.axis_index("c").astype(jnp.int32)
  sid = lax.axis_index("s").astype(jnp.int32)
  wid = sid * jnp.int32(NC) + cid

  _fill_rows(rows, 128, 0.0)
  for q in range(RPT // 128):
    pltpu.sync_copy(rows, acc.at[pl.ds(sid * jnp.int32(RPT) + jnp.int32(q * 128), 128)])
  plsc.subcore_barrier()

  pltpu.sync_copy(src_hbm.at[wid], srcv)
  pltpu.sync_copy(dst_hbm.at[wid], dstv)
  kc = jnp.where(cid == 0, jnp.int32(KC0), jnp.int32(KC1))

  @pl.loop(jnp.int32(0), kc)
  def _(j):
    pltpu.async_copy(hs_hbm.at[srcv.at[j]], rows, sem).wait()
    pltpu.sync_copy(rows, acc.at[dstv.at[j]], add=True)
  plsc.subcore_barrier()
  pltpu.sync_copy(acc.at[pl.ds(sid * jnp.int32(RPT), RPT)],
                  out_hbm.at[pl.ds(cid * jnp.int32(NPAD) + sid * jnp.int32(RPT), RPT)])


def _dot(a, b):
  return jnp.dot(a, b, precision=lax.Precision.HIGHEST,
                 preferred_element_type=jnp.float32)


def _tc_first_body(x_ref, w_ref, d0_ref, d1_ref, hs_ref, dinv_ref):
  deg = 1.0 + d0_ref[:, 0:1] + d1_ref[:, 0:1]
  dinv = lax.rsqrt(deg)
  hs_ref[...] = dinv * _dot(x_ref[...], w_ref[...])
  dinv_ref[...] = jnp.broadcast_to(dinv, (BLK, D))


def _tc_first(x_pad, W1, d0, d1):
  row = pl.BlockSpec((BLK, D), lambda i: (i, jnp.int32(0)))
  deg_spec = pl.BlockSpec((BLK, 16), lambda i: (i, jnp.int32(0)))
  full = pl.BlockSpec((D, D), lambda i: (jnp.int32(0), jnp.int32(0)))
  return pl.pallas_call(
      _tc_first_body,
      grid=(NBLK,),
      in_specs=[row, full, deg_spec, deg_spec],
      out_specs=[row, row],
      out_shape=[jax.ShapeDtypeStruct((NPAD, D), jnp.float32),
                 jax.ShapeDtypeStruct((NPAD, D), jnp.float32)],
  )(x_pad, W1, d0, d1)


def _tc_mid_body(hs_ref, p0_ref, p1_ref, dinv_ref, b_ref, w_ref, out_ref):
  agg = dinv_ref[...] * (p0_ref[...] + p1_ref[...] + hs_ref[...])
  h = jax.nn.relu(agg + b_ref[...])
  out_ref[...] = dinv_ref[...] * _dot(h, w_ref[...])


def _tc_mid(hs, p0, p1, dinv, b_prev, W_next):
  row = pl.BlockSpec((BLK, D), lambda i: (i, jnp.int32(0)))
  full = pl.BlockSpec((D, D), lambda i: (jnp.int32(0), jnp.int32(0)))
  bspec = pl.BlockSpec((1, D), lambda i: (jnp.int32(0), jnp.int32(0)))
  return pl.pallas_call(
      _tc_mid_body,
      grid=(NBLK,),
      in_specs=[row, row, row, row, bspec, full],
      out_specs=row,
      out_shape=jax.ShapeDtypeStruct((NPAD, D), jnp.float32),
  )(hs, p0, p1, dinv, b_prev, W_next)


def _tc_final_body(hs_ref, p0_ref, p1_ref, dinv_ref, b_ref, batch_ref,
                   wl_ref, bl_ref, out_ref, pooled, counts):
  i = pl.program_id(0)
  agg = dinv_ref[...] * (p0_ref[...] + p1_ref[...] + hs_ref[...])
  h = jax.nn.relu(agg + b_ref[...])
  gids = lax.broadcasted_iota(jnp.int32, (BLK, G), 1)
  onehot = (batch_ref[...] == gids).astype(jnp.float32)

  @pl.when(i == 0)
  def _():
    pooled[...] = jnp.zeros((G, D), jnp.float32)
    counts[...] = jnp.zeros((G, D), jnp.float32)

  contract = (((0,), (0,)), ((), ()))
  pooled[...] += lax.dot_general(onehot, h, contract,
                                 precision=lax.Precision.HIGHEST,
                                 preferred_element_type=jnp.float32)
  counts[...] += lax.dot_general(onehot, jnp.ones((BLK, D), jnp.float32),
                                 contract,
                                 precision=lax.Precision.HIGHEST,
                                 preferred_element_type=jnp.float32)

  @pl.when(i == NBLK - 1)
  def _():
    mean = pooled[...] / jnp.maximum(counts[...], 1.0)
    out_ref[...] = _dot(mean, wl_ref[...]) + bl_ref[...]


def _tc_final(hs, p0, p1, dinv, b_prev, batch2d, Wl_pad, bl_pad):
  row = pl.BlockSpec((BLK, D), lambda i: (i, jnp.int32(0)))
  full = pl.BlockSpec((D, D), lambda i: (jnp.int32(0), jnp.int32(0)))
  bspec = pl.BlockSpec((1, D), lambda i: (jnp.int32(0), jnp.int32(0)))
  bat = pl.BlockSpec((BLK, 1), lambda i: (i, jnp.int32(0)))
  outs = pl.BlockSpec((G, D), lambda i: (jnp.int32(0), jnp.int32(0)))
  return pl.pallas_call(
      _tc_final_body,
      grid=(NBLK,),
      in_specs=[row, row, row, row, bspec, bat, full, bspec],
      out_specs=outs,
      out_shape=jax.ShapeDtypeStruct((G, D), jnp.float32),
      scratch_shapes=[pltpu.VMEM((G, D), jnp.float32),
                      pltpu.VMEM((G, D), jnp.float32)],
      compiler_params=pltpu.CompilerParams(
          dimension_semantics=("arbitrary",)),
  )(hs, p0, p1, dinv, b_prev, batch2d, Wl_pad, bl_pad)


def kernel(x, edge_index, batch, W1, b1, W2, b2, W3, b3, Wl, bl):
  src = edge_index[0].astype(jnp.int32)
  dst = edge_index[1].astype(jnp.int32)
  pad_idx = jnp.full((ECAP - E,), N, jnp.int32)

  def shard(idx):
    idx = jnp.concatenate([idx, pad_idx])
    part0 = idx[:E0].reshape(NS, KC0, 128)
    part0 = jnp.concatenate(
        [part0, jnp.full((NS, KCH - KC0, 128), N, jnp.int32)], axis=1)
    part1 = idx[E0:].reshape(NS, KC1, 128)
    return jnp.stack([part0, part1], axis=1).reshape(NW, KCH, 128)

  src_sh = shard(src)
  dst_sh = shard(dst)

  x_pad = jnp.zeros((NPAD, D), jnp.float32).at[:N].set(x)
  batch2d = jnp.concatenate(
      [batch.astype(jnp.int32), jnp.full((NPAD - N,), G, jnp.int32)]
  ).reshape(NPAD, 1)
  b1r, b2r, b3r = b1.reshape(1, D), b2.reshape(1, D), b3.reshape(1, D)
  Wl_pad = jnp.zeros((D, D), jnp.float32).at[:, :DOUT].set(Wl)
  bl_pad = jnp.zeros((1, D), jnp.float32).at[0, :DOUT].set(bl)

  degp = _sc_degree(dst_sh)
  d0, d1 = degp[:NPAD], degp[NPAD:]

  hs1, dinv = _tc_first(x_pad, W1, d0, d1)
  p1 = _sc_scatter(hs1, src_sh, dst_sh)
  hs2 = _tc_mid(hs1, p1[:NPAD], p1[NPAD:], dinv, b1r, W2)
  p2 = _sc_scatter(hs2, src_sh, dst_sh)
  hs3 = _tc_mid(hs2, p2[:NPAD], p2[NPAD:], dinv, b2r, W3)
  p3 = _sc_scatter(hs3, src_sh, dst_sh)
  outp = _tc_final(hs3, p3[:NPAD], p3[NPAD:], dinv, b3r, batch2d,
                   Wl_pad, bl_pad)
  return outp[:G, :DOUT]
